# Initial kernel scaffold; baseline (speedup 1.0000x reference)
import jax
import jax.numpy as jnp
from jax.experimental import pallas as pl


def kernel(x, wq, wkv, wo, bo, rel_h, rel_w, w_conv, b_conv):
    raise NotImplementedError("write your pallas kernel here")


# speedup vs baseline: 1.2363x; 1.2363x over previous
import jax
import jax.numpy as jnp
from jax.experimental import pallas as pl
from jax.experimental.pallas import tpu as pltpu

_BS, _HALO, _HEADS = 8, 4, 4
_R = _BS + 2 * _HALO
_D = 64
_NH = 8


def _strip_kernel(s1_ref, s2_ref, wq_ref, wkv_ref, wo_ref, bo_ref,
                  rwm_ref, rhm_ref, ew_ref, eh_ref, wc_ref, bc_ref, out_ref):
    C = 256
    i = pl.program_id(0) % _NH
    scale = _D ** -0.5

    strip = jnp.concatenate([s1_ref[0], s2_ref[0]], axis=0)
    pix = strip.reshape(16 * 72, C)
    kv = jnp.dot(pix, wkv_ref[...], preferred_element_type=jnp.float32)
    kvw = kv.reshape(16, 72, 2 * C)

    qg = strip[4:12, 4:68, :].reshape(512, C)
    q = jnp.dot(qg, wq_ref[...], preferred_element_type=jnp.float32) * scale

    lwf = []
    lhf = []
    for h in range(_HEADS):
        qh = q[:, h * _D:(h + 1) * _D]
        qr = qh.reshape(8, 8, 8, _D)
        wp = []
        for y in range(8):
            qy = qr[:, :, y, :].reshape(64, _D)
            wp.append(jnp.dot(qy, rwm_ref[:, y * 16:(y + 1) * 16],
                              preferred_element_type=jnp.float32).reshape(8, 8, 16))
        lw = jnp.stack(wp, axis=2).reshape(512, 16)
        hp = []
        for xx in range(8):
            qx = qr[xx].reshape(64, _D)
            hp.append(jnp.dot(qx, rhm_ref[:, xx * 16:(xx + 1) * 16],
                              preferred_element_type=jnp.float32).reshape(1, 8, 8, 16))
        lh = jnp.concatenate(hp, axis=0).reshape(512, 16)
        lwf.append(jnp.dot(lw, ew_ref[...], preferred_element_type=jnp.float32))
        lhf.append(jnp.dot(lh, eh_ref[...], preferred_element_type=jnp.float32))

    lane = jax.lax.broadcasted_iota(jnp.int32, (1, 256), 1)
    ki, kj = lane // 16, lane % 16
    prow = 8 * i + ki
    rvalid = (prow >= 4) & (prow < 68)
    neg = -jnp.finfo(jnp.float32).max

    outs = []
    for j in range(_NH):
        win = kvw[:, 8 * j:8 * j + 16, :].reshape(256, 2 * C)
        pcol = 8 * j + kj
        valid = rvalid & (pcol >= 4) & (pcol < 68)
        heads_out = []
        for h in range(_HEADS):
            qb = lwf[h].reshape(8, 8, 8, 256)[:, j].reshape(64, 256)
            hb = lhf[h].reshape(8, 8, 8, 256)[:, j].reshape(64, 256)
            qh = q[:, h * _D:(h + 1) * _D].reshape(8, 8, 8, _D)[:, j].reshape(64, _D)
            kh = win[:, h * _D:(h + 1) * _D]
            vh = win[:, C + h * _D:C + (h + 1) * _D]
            sim = jax.lax.dot_general(qh, kh, (((1,), (1,)), ((), ())),
                                      preferred_element_type=jnp.float32)
            sim = sim + qb + hb
            sim = jnp.where(valid, sim, neg)
            m = jnp.max(sim, axis=-1, keepdims=True)
            p = jnp.exp(sim - m)
            ssum = jnp.sum(p, axis=-1, keepdims=True)
            attn = p / ssum
            heads_out.append(jnp.dot(attn, vh, preferred_element_type=jnp.float32))
        outs.append(jnp.concatenate(heads_out, axis=1).reshape(8, 8, 256))

    y_strip = jnp.stack(outs, axis=1).reshape(512, 256)
    y_attn = jnp.dot(y_strip, wo_ref[...],
                     preferred_element_type=jnp.float32) + bo_ref[...]
    conv = jnp.dot(y_attn, wc_ref[...],
                   preferred_element_type=jnp.float32) + bc_ref[...]
    out_ref[...] = conv.reshape(1, 1, 8, 64, 1024)


def kernel(x, wq, wkv, wo, bo, rel_h, rel_w, w_conv, b_conv):
    B, C, H, W = x.shape
    nh = H // _BS

    xt = jnp.transpose(x, (0, 2, 3, 1))
    xp = jnp.pad(xt, ((0, 0), (_HALO, _HALO), (_HALO, _HALO), (0, 0)))

    ry = jnp.arange(_R)[None, :] - jnp.arange(_BS)[:, None] + (_R - 1)
    rwm = jnp.transpose(rel_w[ry], (2, 0, 1)).reshape(_D, 128)
    rhm = jnp.transpose(rel_h[ry], (2, 0, 1)).reshape(_D, 128)

    lane = jnp.arange(256)
    ew = (lane[None, :] % 16 == jnp.arange(16)[:, None]).astype(jnp.float32)
    eh = (lane[None, :] // 16 == jnp.arange(16)[:, None]).astype(jnp.float32)

    grid = (B * nh,)
    const = lambda s: (0, 0)
    out8 = pl.pallas_call(
        _strip_kernel,
        grid=grid,
        in_specs=[
            pl.BlockSpec((1, _BS, 72, C), lambda s: (s // _NH, s % _NH, 0, 0)),
            pl.BlockSpec((1, _BS, 72, C), lambda s: (s // _NH, s % _NH + 1, 0, 0)),
            pl.BlockSpec((C, C), const),
            pl.BlockSpec((C, 2 * C), const),
            pl.BlockSpec((C, C), const),
            pl.BlockSpec((1, C), const),
            pl.BlockSpec((_D, 128), const),
            pl.BlockSpec((_D, 128), const),
            pl.BlockSpec((16, 256), const),
            pl.BlockSpec((16, 256), const),
            pl.BlockSpec((C, 4 * C), const),
            pl.BlockSpec((1, 4 * C), const),
        ],
        out_specs=pl.BlockSpec((1, 1, _BS, 8 * _NH, 4 * C),
                               lambda s: (s // _NH, s % _NH, 0, 0, 0)),
        out_shape=jax.ShapeDtypeStruct((B, nh, _BS, 8 * _NH, 4 * C), jnp.float32),
        compiler_params=pltpu.CompilerParams(
            dimension_semantics=("parallel",),
            vmem_limit_bytes=50 * 1024 * 1024,
        ),
    )(xp, xp, wq.T, wkv.T, wo.T, bo.reshape(1, C),
      rwm, rhm, ew, eh, w_conv.T, b_conv.reshape(1, 4 * C))

    out = out8.reshape(B, nh, _BS, W, C, 2, 2)
    out = out.transpose(0, 4, 1, 2, 5, 3, 6).reshape(B, C, 2 * H, 2 * W)
    return out
